# Initial kernel scaffold; baseline (speedup 1.0000x reference)
import functools

import jax
import jax.numpy as jnp
from jax import lax
from jax.experimental import pallas as pl
from jax.experimental.pallas import tpu as pltpu
from jax.experimental.pallas import tpu_sc as plsc

N_NODES = 10000
N_EDGES = 320000
D = 128

NC = 2
NS = 16
NW = NC * NS
EPW = N_EDGES // NW
CHUNK = 80
NITER = EPW // CHUNK
RPT = N_NODES // NS
ZROWS = 125


def _agg_body(x_hbm, src_hbm, dst_hbm, psum_hbm, pcnt_hbm,
              acc_sh, cnt_sh, sidx, didx, rows, ones, zacc, zcnt, sem):
    cid = lax.axis_index("c")
    sid = lax.axis_index("s")

    def fill_zacc(i, _):
        for j in range(D // 16):
            zacc[i, pl.ds(j * 16, 16)] = jnp.zeros((16,), jnp.float32)
        return _
    lax.fori_loop(0, ZROWS, fill_zacc, None)

    def fill_zcnt(i, _):
        zcnt[i, :] = jnp.zeros((16,), jnp.float32)
        return _
    lax.fori_loop(0, RPT, fill_zcnt, None)

    def fill_ones(i, _):
        ones[i, :] = jnp.ones((16,), jnp.float32)
        return _
    lax.fori_loop(0, CHUNK, fill_ones, None)

    for k in range(RPT // ZROWS):
        pltpu.sync_copy(zacc, acc_sh.at[pl.ds(sid * RPT + k * ZROWS, ZROWS)])
    pltpu.sync_copy(zcnt, cnt_sh.at[pl.ds(sid * RPT, RPT)])
    plsc.subcore_barrier()

    wid = sid * NC + cid
    ebase = wid * EPW

    def edge_step(it, _):
        off = ebase + it * CHUNK
        pltpu.sync_copy(src_hbm.at[pl.ds(off, CHUNK)], sidx)
        pltpu.sync_copy(dst_hbm.at[pl.ds(off, CHUNK)], didx)
        pltpu.async_copy(x_hbm.at[sidx], rows, sem).wait()
        pltpu.sync_copy(rows, acc_sh.at[didx], add=True)
        pltpu.sync_copy(ones, cnt_sh.at[didx], add=True)
        return _
    lax.fori_loop(0, NITER, edge_step, None)

    plsc.subcore_barrier()

    r0 = sid * RPT
    pltpu.sync_copy(acc_sh.at[pl.ds(r0, RPT)], psum_hbm.at[cid, pl.ds(r0, RPT)])
    pltpu.sync_copy(cnt_sh.at[pl.ds(r0, RPT)], pcnt_hbm.at[cid, pl.ds(r0, RPT)])


_agg = pl.kernel(
    _agg_body,
    out_type=(
        jax.ShapeDtypeStruct((NC, N_NODES, D), jnp.float32),
        jax.ShapeDtypeStruct((NC, N_NODES, 16), jnp.float32),
    ),
    mesh=plsc.VectorSubcoreMesh(
        core_axis_name="c", subcore_axis_name="s", num_cores=NC, num_subcores=NS
    ),
    scratch_types=[
        pltpu.VMEM_SHARED((N_NODES, D), jnp.float32),
        pltpu.VMEM_SHARED((N_NODES, 16), jnp.float32),
        pltpu.VMEM((CHUNK,), jnp.int32),
        pltpu.VMEM((CHUNK,), jnp.int32),
        pltpu.VMEM((CHUNK, D), jnp.float32),
        pltpu.VMEM((CHUNK, 16), jnp.float32),
        pltpu.VMEM((ZROWS, D), jnp.float32),
        pltpu.VMEM((RPT, 16), jnp.float32),
        pltpu.SemaphoreType.DMA,
    ],
)


BLK = 1000


def _dense_body(psum_ref, pcnt_ref, x_ref, wl_ref, wr_ref, b_ref, g_ref,
                be_ref, out_ref):
    s = psum_ref[0] + psum_ref[1]
    c = pcnt_ref[0, :, 0:1] + pcnt_ref[1, :, 0:1]
    mean = s / jnp.maximum(c, 1.0)
    dn = (((1,), (1,)), ((), ()))
    h = (lax.dot_general(mean, wl_ref[...], dn, preferred_element_type=jnp.float32)
         + lax.dot_general(x_ref[...], wr_ref[...], dn, preferred_element_type=jnp.float32)
         + b_ref[...])
    mu = jnp.mean(h, axis=1, keepdims=True)
    var = jnp.mean(jnp.square(h - mu), axis=1, keepdims=True)
    hn = (h - mu) * lax.rsqrt(var + 1e-5) * g_ref[...] + be_ref[...]
    out_ref[...] = jnp.where(hn >= 0, hn, 0.01 * hn)


_dense = pl.pallas_call(
    _dense_body,
    grid=(N_NODES // BLK,),
    in_specs=[
        pl.BlockSpec((NC, BLK, D), lambda i: (0, i, 0)),
        pl.BlockSpec((NC, BLK, 16), lambda i: (0, i, 0)),
        pl.BlockSpec((BLK, D), lambda i: (i, 0)),
        pl.BlockSpec((D, D), lambda i: (0, 0)),
        pl.BlockSpec((D, D), lambda i: (0, 0)),
        pl.BlockSpec((1, D), lambda i: (0, 0)),
        pl.BlockSpec((1, D), lambda i: (0, 0)),
        pl.BlockSpec((1, D), lambda i: (0, 0)),
    ],
    out_specs=pl.BlockSpec((BLK, D), lambda i: (i, 0)),
    out_shape=jax.ShapeDtypeStruct((N_NODES, D), jnp.float32),
)


@jax.jit
def kernel(x, edge_index, W_l, b_l, W_r, gamma, beta):
    src = edge_index[0].astype(jnp.int32)
    dst = edge_index[1].astype(jnp.int32)
    psum, pcnt = _agg(x, src, dst)
    return _dense(psum, pcnt, x, W_l, W_r,
                  b_l.reshape(1, D), gamma.reshape(1, D), beta.reshape(1, D))


# speedup vs baseline: 6.1740x; 6.1740x over previous
import functools

import jax
import jax.numpy as jnp
from jax import lax
from jax.experimental import pallas as pl
from jax.experimental.pallas import tpu as pltpu
from jax.experimental.pallas import tpu_sc as plsc

N_NODES = 10000
N_EDGES = 320000
D = 128

NC = 2
NS = 16
NW = NC * NS
EPW = N_EDGES // NW
CHUNK = 80
NITER = EPW // CHUNK
ACC = 10240
RPT = ACC // NS
ZROWS = 128
CR = ACC // D
CRPT = CR // NS


def _agg_body(x_hbm, src_hbm, dst_hbm, psum_hbm, pcnt_hbm,
              acc_sh, cnt_sh, sidx, didx, rows, hist, rowids, zacc, sem):
    cid = lax.axis_index("c")
    sid = lax.axis_index("s")

    def fill_zacc(i, _):
        for j in range(D // 16):
            zacc[i, pl.ds(j * 16, 16)] = jnp.zeros((16,), jnp.float32)
        return _
    lax.fori_loop(0, ZROWS, fill_zacc, None)

    def fill_hist(i, _):
        for j in range(D // 16):
            hist[i, pl.ds(j * 16, 16)] = jnp.zeros((16,), jnp.float32)
        return _
    lax.fori_loop(0, CR, fill_hist, None)

    for k in range(CR // 16):
        rowids[pl.ds(k * 16, 16)] = lax.iota(jnp.int32, 16) + (k * 16)

    for k in range(RPT // ZROWS):
        pltpu.sync_copy(zacc, acc_sh.at[pl.ds(sid * RPT + k * ZROWS, ZROWS)])

    @pl.when(sid == 0)
    def _():
        pltpu.sync_copy(hist, cnt_sh)
    plsc.subcore_barrier()

    wid = sid * NC + cid
    ebase = wid * EPW

    def edge_step(it, _):
        off = ebase + it * CHUNK
        pltpu.sync_copy(src_hbm.at[pl.ds(off, CHUNK)], sidx)
        pltpu.sync_copy(dst_hbm.at[pl.ds(off, CHUNK)], didx)
        pltpu.async_copy(x_hbm.at[sidx], rows, sem).wait()
        pltpu.sync_copy(rows, acc_sh.at[didx], add=True)
        for k in range(CHUNK // 16):
            dv = didx[pl.ds(k * 16, 16)]
            hi = lax.shift_right_logical(dv, 7)
            lo = lax.bitwise_and(dv, 127)
            plsc.addupdate_scatter(hist, [hi, lo], jnp.ones((16,), jnp.float32))
        return _
    lax.fori_loop(0, NITER, edge_step, None)

    plsc.subcore_barrier()

    pltpu.sync_copy(hist, cnt_sh.at[rowids], add=True)
    plsc.subcore_barrier()

    r0 = sid * RPT
    pltpu.sync_copy(acc_sh.at[pl.ds(r0, RPT)], psum_hbm.at[cid, pl.ds(r0, RPT)])
    pltpu.sync_copy(cnt_sh.at[pl.ds(sid * CRPT, CRPT)],
                    pcnt_hbm.at[cid, pl.ds(sid * CRPT, CRPT)])


@functools.cache
def _agg():
  return pl.kernel(
    _agg_body,
    out_type=(
        jax.ShapeDtypeStruct((NC, ACC, D), jnp.float32),
        jax.ShapeDtypeStruct((NC, CR, D), jnp.float32),
    ),
    mesh=plsc.VectorSubcoreMesh(
        core_axis_name="c", subcore_axis_name="s", num_cores=NC, num_subcores=NS
    ),
    scratch_types=[
        pltpu.VMEM_SHARED((ACC, D), jnp.float32),
        pltpu.VMEM_SHARED((CR, D), jnp.float32),
        pltpu.VMEM((CHUNK,), jnp.int32),
        pltpu.VMEM((CHUNK,), jnp.int32),
        pltpu.VMEM((CHUNK, D), jnp.float32),
        pltpu.VMEM((CR, D), jnp.float32),
        pltpu.VMEM((CR,), jnp.int32),
        pltpu.VMEM((ZROWS, D), jnp.float32),
        pltpu.SemaphoreType.DMA,
    ],
    compiler_params=pltpu.CompilerParams(use_tc_tiling_on_sc=False,
                                         needs_layout_passes=False),
  )


BLK = 1024


def _dense_body(psum_ref, pcnt_ref, x_ref, wl_ref, wr_ref, b_ref, g_ref,
                be_ref, out_ref):
    s = psum_ref[0] + psum_ref[1]
    c = pcnt_ref[0] + pcnt_ref[1]
    mean = s / jnp.maximum(c, 1.0)
    dn = (((1,), (1,)), ((), ()))
    h = (lax.dot_general(mean, wl_ref[...], dn, preferred_element_type=jnp.float32)
         + lax.dot_general(x_ref[...], wr_ref[...], dn, preferred_element_type=jnp.float32)
         + b_ref[...])
    mu = jnp.mean(h, axis=1, keepdims=True)
    var = jnp.mean(jnp.square(h - mu), axis=1, keepdims=True)
    hn = (h - mu) * lax.rsqrt(var + 1e-5) * g_ref[...] + be_ref[...]
    out_ref[...] = jnp.where(hn >= 0, hn, 0.01 * hn)


_dense = pl.pallas_call(
    _dense_body,
    grid=(ACC // BLK,),
    in_specs=[
        pl.BlockSpec((NC, BLK, D), lambda i: (0, i, 0)),
        pl.BlockSpec((NC, BLK, 1), lambda i: (0, i, 0)),
        pl.BlockSpec((BLK, D), lambda i: (i, 0)),
        pl.BlockSpec((D, D), lambda i: (0, 0)),
        pl.BlockSpec((D, D), lambda i: (0, 0)),
        pl.BlockSpec((1, D), lambda i: (0, 0)),
        pl.BlockSpec((1, D), lambda i: (0, 0)),
        pl.BlockSpec((1, D), lambda i: (0, 0)),
    ],
    out_specs=pl.BlockSpec((BLK, D), lambda i: (i, 0)),
    out_shape=jax.ShapeDtypeStruct((ACC, D), jnp.float32),
)


@jax.jit
def kernel(x, edge_index, W_l, b_l, W_r, gamma, beta):
    src = edge_index[0].astype(jnp.int32)
    dst = edge_index[1].astype(jnp.int32)
    psum, pcnt = _agg()(x, src, dst)
    cnt_col = pcnt.reshape(NC, ACC, 1)
    xp = jnp.pad(x, ((0, ACC - N_NODES), (0, 0)))
    out = _dense(psum, cnt_col, xp, W_l, W_r,
                 b_l.reshape(1, D), gamma.reshape(1, D), beta.reshape(1, D))
    return out[:N_NODES]


# speedup vs baseline: 10.6603x; 1.7267x over previous
import functools

import jax
import jax.numpy as jnp
from jax import lax
from jax.experimental import pallas as pl
from jax.experimental.pallas import tpu as pltpu
from jax.experimental.pallas import tpu_sc as plsc

N_NODES = 10000
N_EDGES = 320000
D = 128

NC = 2
NS = 16
NW = NC * NS
EPW = N_EDGES // NW
CHUNK = 80
NITER = EPW // CHUNK
ACC = 10240
RPT = ACC // NS
ZROWS = 128
CR = ACC // D
CRPT = CR // NS


def _agg_body(x_hbm, src_hbm, dst_hbm, psum_hbm, pcnt_hbm,
              acc_sh, cnt_sh, sbuf, didx, rows, hist, rowids,
              semg0, semg1, semi0, semi1):
    cid = lax.axis_index("c")
    sid = lax.axis_index("s")
    semg = (semg0, semg1)
    semi = (semi0, semi1)

    def fill_zero(i, _):
        for j in range(D // 16):
            rows[0, i, pl.ds(j * 16, 16)] = jnp.zeros((16,), jnp.float32)
            hist[i, pl.ds(j * 16, 16)] = jnp.zeros((16,), jnp.float32)
        return _
    lax.fori_loop(0, CR, fill_zero, None)

    for k in range(CR // 16):
        rowids[pl.ds(k * 16, 16)] = lax.iota(jnp.int32, 16) + (k * 16)

    for k in range(RPT // CHUNK):
        pltpu.sync_copy(rows.at[0], acc_sh.at[pl.ds(sid * RPT + k * CHUNK, CHUNK)])

    @pl.when(sid == 0)
    def _():
        pltpu.sync_copy(rows.at[0], cnt_sh)
    plsc.subcore_barrier()

    wid = sid * NC + cid
    pltpu.sync_copy(dst_hbm.at[wid], didx)
    pltpu.async_copy(src_hbm.at[wid, 0], sbuf.at[0], semi0)
    pltpu.async_copy(src_hbm.at[wid, 1], sbuf.at[1], semi1)
    pltpu.make_async_copy(src_hbm.at[wid, 0], sbuf.at[0], semi0).wait()
    pltpu.async_copy(x_hbm.at[sbuf.at[0]], rows.at[0], semg0)

    def chunk_tail(j, b):
        for k in range(CHUNK // 16):
            dv = didx[j, pl.ds(k * 16, 16)]
            hi = lax.shift_right_logical(dv, 7)
            lo = lax.bitwise_and(dv, 127)
            plsc.addupdate_scatter(hist, [hi, lo], jnp.ones((16,), jnp.float32))
        pltpu.sync_copy(rows.at[b], acc_sh.at[didx.at[j]], add=True)

    def edge_step(t, _):
        for b in range(2):
            j = 2 * t + b
            pltpu.make_async_copy(x_hbm.at[sbuf.at[b]], rows.at[b],
                                  semg[b]).wait()

            @pl.when(j + 2 < NITER)
            def _():
                pltpu.async_copy(src_hbm.at[wid, j + 2], sbuf.at[b], semi[b])

            pltpu.make_async_copy(src_hbm.at[wid, j + 1], sbuf.at[1 - b],
                                  semi[1 - b]).wait()
            pltpu.async_copy(x_hbm.at[sbuf.at[1 - b]], rows.at[1 - b],
                             semg[1 - b])
            chunk_tail(j, b)
        return _
    lax.fori_loop(0, (NITER - 1) // 2, edge_step, None)
    pltpu.make_async_copy(x_hbm.at[sbuf.at[0]], rows.at[0], semg0).wait()
    chunk_tail(NITER - 1, 0)

    plsc.subcore_barrier()

    pltpu.sync_copy(hist, cnt_sh.at[rowids], add=True)
    plsc.subcore_barrier()

    r0 = sid * RPT
    pltpu.sync_copy(acc_sh.at[pl.ds(r0, RPT)], psum_hbm.at[cid, pl.ds(r0, RPT)])
    pltpu.sync_copy(cnt_sh.at[pl.ds(sid * CRPT, CRPT)],
                    pcnt_hbm.at[cid, pl.ds(sid * CRPT, CRPT)])


@functools.cache
def _agg():
  return pl.kernel(
    _agg_body,
    out_type=(
        jax.ShapeDtypeStruct((NC, ACC, D), jnp.float32),
        jax.ShapeDtypeStruct((NC, CR, D), jnp.float32),
    ),
    mesh=plsc.VectorSubcoreMesh(
        core_axis_name="c", subcore_axis_name="s", num_cores=NC, num_subcores=NS
    ),
    scratch_types=[
        pltpu.VMEM_SHARED((ACC, D), jnp.float32),
        pltpu.VMEM_SHARED((CR, D), jnp.float32),
        pltpu.VMEM((2, CHUNK), jnp.int32),
        pltpu.VMEM((NITER, CHUNK), jnp.int32),
        pltpu.VMEM((2, CHUNK, D), jnp.float32),
        pltpu.VMEM((CR, D), jnp.float32),
        pltpu.VMEM((CR,), jnp.int32),
        pltpu.SemaphoreType.DMA,
        pltpu.SemaphoreType.DMA,
        pltpu.SemaphoreType.DMA,
        pltpu.SemaphoreType.DMA,
    ],
    compiler_params=pltpu.CompilerParams(use_tc_tiling_on_sc=False,
                                         needs_layout_passes=False),
  )


BLK = 1024


def _dense_body(psum_ref, pcnt_ref, x_ref, wl_ref, wr_ref, b_ref, g_ref,
                be_ref, out_ref):
    s = psum_ref[0] + psum_ref[1]
    c = pcnt_ref[0] + pcnt_ref[1]
    mean = s / jnp.maximum(c, 1.0)
    dn = (((1,), (1,)), ((), ()))
    h = (lax.dot_general(mean, wl_ref[...], dn, preferred_element_type=jnp.float32)
         + lax.dot_general(x_ref[...], wr_ref[...], dn, preferred_element_type=jnp.float32)
         + b_ref[...])
    mu = jnp.mean(h, axis=1, keepdims=True)
    var = jnp.mean(jnp.square(h - mu), axis=1, keepdims=True)
    hn = (h - mu) * lax.rsqrt(var + 1e-5) * g_ref[...] + be_ref[...]
    out_ref[...] = jnp.where(hn >= 0, hn, 0.01 * hn)


_dense = pl.pallas_call(
    _dense_body,
    grid=(ACC // BLK,),
    in_specs=[
        pl.BlockSpec((NC, BLK, D), lambda i: (0, i, 0)),
        pl.BlockSpec((NC, BLK, 1), lambda i: (0, i, 0)),
        pl.BlockSpec((BLK, D), lambda i: (i, 0)),
        pl.BlockSpec((D, D), lambda i: (0, 0)),
        pl.BlockSpec((D, D), lambda i: (0, 0)),
        pl.BlockSpec((1, D), lambda i: (0, 0)),
        pl.BlockSpec((1, D), lambda i: (0, 0)),
        pl.BlockSpec((1, D), lambda i: (0, 0)),
    ],
    out_specs=pl.BlockSpec((BLK, D), lambda i: (i, 0)),
    out_shape=jax.ShapeDtypeStruct((ACC, D), jnp.float32),
)


@jax.jit
def kernel(x, edge_index, W_l, b_l, W_r, gamma, beta):
    src = edge_index[0].astype(jnp.int32).reshape(NW, NITER, CHUNK)
    dst = edge_index[1].astype(jnp.int32).reshape(NW, NITER, CHUNK)
    psum, pcnt = _agg()(x, src, dst)
    cnt_col = pcnt.reshape(NC, ACC, 1)
    xp = jnp.pad(x, ((0, ACC - N_NODES), (0, 0)))
    out = _dense(psum, cnt_col, xp, W_l, W_r,
                 b_l.reshape(1, D), gamma.reshape(1, D), beta.reshape(1, D))
    return out[:N_NODES]


# speedup vs baseline: 17.7831x; 1.6682x over previous
import functools

import jax
import jax.numpy as jnp
from jax import lax
from jax.experimental import pallas as pl
from jax.experimental.pallas import tpu as pltpu
from jax.experimental.pallas import tpu_sc as plsc

N_NODES = 10000
N_EDGES = 320000
D = 128

NC = 2
NS = 16
NW = NC * NS
EPW = N_EDGES // NW
CHUNK = 80
NITER = EPW // CHUNK
NBUF = 3
ACC = 10240
RPT = ACC // NS
CR = ACC // D
CRPT = CR // NS


def _agg_body(x_hbm, ei_hbm, psum_hbm, pcnt_hbm,
              acc_sh, cnt_sh, sbuf, dbuf, rows, hist, rowids,
              semg0, semg1, semg2, semi0, semi1, semi2,
              semd0, semd1, semd2, semc0, semc1, semc2):
    cid = lax.axis_index("c")
    sid = lax.axis_index("s")
    semg = (semg0, semg1, semg2)
    semi = (semi0, semi1, semi2)
    semd = (semd0, semd1, semd2)
    semc = (semc0, semc1, semc2)

    def fill_zero(i, _):
        for j in range(D // 16):
            rows[0, i, pl.ds(j * 16, 16)] = jnp.zeros((16,), jnp.float32)
        return _
    lax.fori_loop(0, CHUNK, fill_zero, None)

    def fill_hist(i, _):
        for j in range(D // 16):
            hist[i, pl.ds(j * 16, 16)] = jnp.zeros((16,), jnp.float32)
        return _
    lax.fori_loop(0, CR, fill_hist, None)

    for k in range(CR // 16):
        rowids[pl.ds(k * 16, 16)] = lax.iota(jnp.int32, 16) + (k * 16)

    for k in range(RPT // CHUNK):
        pltpu.sync_copy(rows.at[0], acc_sh.at[pl.ds(sid * RPT + k * CHUNK, CHUNK)])

    @pl.when(sid == 0)
    def _():
        pltpu.sync_copy(rows.at[0, pl.ds(0, CR)], cnt_sh)
    plsc.subcore_barrier()

    wid = sid * NC + cid
    ebase = wid * EPW

    def sidx_load(j, b):
        pltpu.async_copy(ei_hbm.at[0, pl.ds(ebase + j * CHUNK, CHUNK)],
                         sbuf.at[b], semi[b])

    def didx_load(j, b):
        pltpu.async_copy(ei_hbm.at[1, pl.ds(ebase + j * CHUNK, CHUNK)],
                         dbuf.at[b], semd[b])

    def hist_update(b):
        for k in range(CHUNK // 16):
            dv = dbuf[b, pl.ds(k * 16, 16)]
            hi = lax.shift_right_logical(dv, 7)
            lo = lax.bitwise_and(dv, 127)
            plsc.addupdate_scatter(hist, [hi, lo], jnp.ones((16,), jnp.float32))

    def gather_fire(b):
        pltpu.async_copy(x_hbm.at[sbuf.at[b]], rows.at[b], semg[b])

    def gather_wait(b):
        pltpu.make_async_copy(x_hbm.at[sbuf.at[b]], rows.at[b], semg[b]).wait()

    def idx_wait(sem, buf, b):
        pltpu.make_async_copy(ei_hbm.at[0, pl.ds(ebase, CHUNK)], buf.at[b],
                              sem[b]).wait()

    def scatter_wait(b):
        pltpu.make_async_copy(rows.at[b], acc_sh.at[dbuf.at[b]],
                              semc[b]).wait()

    for b in range(NBUF):
        sidx_load(b, b)
    didx_load(0, 0)
    didx_load(1, 1)
    idx_wait(semi, sbuf, 0)
    gather_fire(0)
    idx_wait(semi, sbuf, 1)
    gather_fire(1)

    def iter_body(j, b, first, has2, has3):
        bn = (b + 2) % 3
        gather_wait(b)
        if has3:
            sidx_load(j + 3, b)
        if not first:
            scatter_wait(bn)
        if has2:
            didx_load(j + 2, bn)
            idx_wait(semi, sbuf, bn)
            gather_fire(bn)
        idx_wait(semd, dbuf, b)
        pltpu.async_copy(rows.at[b], acc_sh.at[dbuf.at[b]], semc[b], add=True)
        hist_update(b)

    iter_body(0, 0, True, True, True)

    def edge_step(t, _):
        iter_body(3 * t + 1, 1, False, True, True)
        iter_body(3 * t + 2, 2, False, True, True)
        iter_body(3 * t + 3, 0, False, True, True)
        return _
    lax.fori_loop(0, (NITER - 5) // 3, edge_step, None)
    iter_body(NITER - 4, 1, False, True, True)
    iter_body(NITER - 3, 2, False, True, False)
    iter_body(NITER - 2, 0, False, False, False)
    iter_body(NITER - 1, 1, False, False, False)

    scatter_wait(1)

    plsc.subcore_barrier()

    pltpu.sync_copy(hist, cnt_sh.at[rowids], add=True)
    plsc.subcore_barrier()

    r0 = sid * RPT
    pltpu.sync_copy(acc_sh.at[pl.ds(r0, RPT)], psum_hbm.at[cid, pl.ds(r0, RPT)])
    pltpu.sync_copy(cnt_sh.at[pl.ds(sid * CRPT, CRPT)],
                    pcnt_hbm.at[cid, pl.ds(sid * CRPT, CRPT)])


@functools.cache
def _agg():
  return pl.kernel(
    _agg_body,
    out_type=(
        jax.ShapeDtypeStruct((NC, ACC, D), jnp.float32),
        jax.ShapeDtypeStruct((NC, CR, D), jnp.float32),
    ),
    mesh=plsc.VectorSubcoreMesh(
        core_axis_name="c", subcore_axis_name="s", num_cores=NC, num_subcores=NS
    ),
    scratch_types=[
        pltpu.VMEM_SHARED((ACC, D), jnp.float32),
        pltpu.VMEM_SHARED((CR, D), jnp.float32),
        pltpu.VMEM((NBUF, CHUNK), jnp.int32),
        pltpu.VMEM((NBUF, CHUNK), jnp.int32),
        pltpu.VMEM((NBUF, CHUNK, D), jnp.float32),
        pltpu.VMEM((CR, D), jnp.float32),
        pltpu.VMEM((CR,), jnp.int32),
        pltpu.SemaphoreType.DMA,
        pltpu.SemaphoreType.DMA,
        pltpu.SemaphoreType.DMA,
        pltpu.SemaphoreType.DMA,
        pltpu.SemaphoreType.DMA,
        pltpu.SemaphoreType.DMA,
        pltpu.SemaphoreType.DMA,
        pltpu.SemaphoreType.DMA,
        pltpu.SemaphoreType.DMA,
        pltpu.SemaphoreType.DMA,
        pltpu.SemaphoreType.DMA,
        pltpu.SemaphoreType.DMA,
    ],
    compiler_params=pltpu.CompilerParams(use_tc_tiling_on_sc=False,
                                         needs_layout_passes=False),
  )


BLK = 2048


def _dense_body(psum_ref, pcnt_ref, x_ref, wl_ref, wr_ref, b_ref, g_ref,
                be_ref, out_ref):
    s = psum_ref[0] + psum_ref[1]
    c8 = pcnt_ref[0] + pcnt_ref[1]
    crb = CR // (ACC // BLK)
    row = lax.broadcasted_iota(jnp.int32, (BLK, crb), 0) // D
    sel = (row == lax.broadcasted_iota(jnp.int32, (BLK, crb), 1)).astype(jnp.float32)
    t = lax.dot_general(sel, c8, (((1,), (0,)), ((), ())),
                        preferred_element_type=jnp.float32)
    lane = lax.broadcasted_iota(jnp.int32, (BLK, D), 0) % D
    msk = lane == lax.broadcasted_iota(jnp.int32, (BLK, D), 1)
    c = jnp.sum(jnp.where(msk, t, 0.0), axis=1, keepdims=True)
    mean = s / jnp.maximum(c, 1.0)
    dn = (((1,), (1,)), ((), ()))
    h = (lax.dot_general(mean, wl_ref[...], dn, preferred_element_type=jnp.float32)
         + lax.dot_general(x_ref[...], wr_ref[...], dn, preferred_element_type=jnp.float32)
         + b_ref[...])
    mu = jnp.mean(h, axis=1, keepdims=True)
    var = jnp.mean(jnp.square(h - mu), axis=1, keepdims=True)
    hn = (h - mu) * lax.rsqrt(var + 1e-5) * g_ref[...] + be_ref[...]
    out_ref[...] = jnp.where(hn >= 0, hn, 0.01 * hn)


_dense = pl.pallas_call(
    _dense_body,
    grid=(ACC // BLK,),
    in_specs=[
        pl.BlockSpec((NC, BLK, D), lambda i: (0, i, 0)),
        pl.BlockSpec((NC, CR // (ACC // BLK), D), lambda i: (0, i, 0)),
        pl.BlockSpec((BLK, D), lambda i: (i, 0)),
        pl.BlockSpec((D, D), lambda i: (0, 0)),
        pl.BlockSpec((D, D), lambda i: (0, 0)),
        pl.BlockSpec((1, D), lambda i: (0, 0)),
        pl.BlockSpec((1, D), lambda i: (0, 0)),
        pl.BlockSpec((1, D), lambda i: (0, 0)),
    ],
    out_specs=pl.BlockSpec((BLK, D), lambda i: (i, 0)),
    out_shape=jax.ShapeDtypeStruct((N_NODES, D), jnp.float32),
)


@jax.jit
def kernel(x, edge_index, W_l, b_l, W_r, gamma, beta):
    ei = edge_index.astype(jnp.int32)
    psum, pcnt = _agg()(x, ei)
    out = _dense(psum, pcnt, x, W_l, W_r,
                 b_l.reshape(1, D), gamma.reshape(1, D), beta.reshape(1, D))
    return out
